# Initial kernel scaffold; baseline (speedup 1.0000x reference)
import functools

import jax
import jax.numpy as jnp
from jax import lax
from jax.experimental import pallas as pl
from jax.experimental.pallas import tpu as pltpu
from jax.experimental.pallas import tpu_sc as plsc

B = 16384
D = 32
NC, NS = 2, 16
NW = NC * NS
B_PER_W = B // NW
CHUNK = 128
CPW = B_PER_W // CHUNK

_sc_mesh = plsc.VectorSubcoreMesh(core_axis_name="c", subcore_axis_name="s")


@functools.partial(
    pl.kernel,
    out_type=(
        jax.ShapeDtypeStruct((B, D), jnp.float32),
        jax.ShapeDtypeStruct((B, D), jnp.float32),
    ),
    mesh=_sc_mesh,
    scratch_types=[
        pltpu.VMEM((CPW, CHUNK), jnp.int32),
        pltpu.VMEM((CPW, CHUNK), jnp.int32),
        pltpu.VMEM((B_PER_W, D), jnp.float32),
        pltpu.VMEM((B_PER_W, D), jnp.float32),
        pltpu.SemaphoreType.DMA,
    ],
)
def _sc_gather(uid_hbm, mid_hbm, ut_hbm, mt_hbm, ue_hbm, me_hbm,
               idx_u, idx_m, rows_u, rows_m, sem):
    wid = lax.axis_index("s") * NC + lax.axis_index("c")
    base = wid * CPW
    pltpu.sync_copy(uid_hbm.at[pl.ds(base, CPW)], idx_u)
    pltpu.sync_copy(mid_hbm.at[pl.ds(base, CPW)], idx_m)
    copies = []
    for j in range(CPW):
        copies.append(pltpu.async_copy(
            ut_hbm.at[idx_u.at[j]], rows_u.at[pl.ds(j * CHUNK, CHUNK)], sem))
        copies.append(pltpu.async_copy(
            mt_hbm.at[idx_m.at[j]], rows_m.at[pl.ds(j * CHUNK, CHUNK)], sem))
    for c in copies:
        c.wait()
    row0 = wid * B_PER_W
    pltpu.sync_copy(rows_u, ue_hbm.at[pl.ds(row0, B_PER_W)])
    pltpu.sync_copy(rows_m, me_hbm.at[pl.ds(row0, B_PER_W)])


BLK = 2048


def _mlp_body(ue_ref, me_ref, w1u_ref, w1m_ref, b1_ref, w2_ref, b2_ref,
              w3_ref, b3_ref, out_ref):
    x1 = jnp.dot(ue_ref[...], w1u_ref[...], preferred_element_type=jnp.float32)
    x2 = jnp.dot(me_ref[...], w1m_ref[...], preferred_element_type=jnp.float32)
    h1 = jnp.maximum(x1 + x2 + b1_ref[...], 0.0)
    h2 = jnp.maximum(
        jnp.dot(h1, w2_ref[...], preferred_element_type=jnp.float32)
        + b2_ref[...], 0.0)
    out_ref[...] = (
        jnp.dot(h2, w3_ref[...], preferred_element_type=jnp.float32)
        + b3_ref[...])


def _mlp(ue, me, w1u, w1m, b1, w2, b2, w3, b3):
    grid = (B // BLK,)
    fixed = lambda shape: pl.BlockSpec(shape, lambda i: (0, 0))
    return pl.pallas_call(
        _mlp_body,
        grid=grid,
        in_specs=[
            pl.BlockSpec((BLK, D), lambda i: (i, 0)),
            pl.BlockSpec((BLK, D), lambda i: (i, 0)),
            fixed((D, 256)),
            fixed((D, 256)),
            fixed((1, 256)),
            fixed((256, 64)),
            fixed((1, 64)),
            fixed((64, 1)),
            fixed((1, 1)),
        ],
        out_specs=pl.BlockSpec((BLK, 1), lambda i: (i, 0)),
        out_shape=jax.ShapeDtypeStruct((B, 1), jnp.float32),
    )(ue, me, w1u, w1m, b1, w2, b2, w3, b3)


def kernel(user_id, movie_title, user_table, movie_table,
           W1, b1, W2, b2, W3, b3):
    uid = user_id.astype(jnp.int32).reshape(NW * CPW, CHUNK)
    mid = movie_title.astype(jnp.int32).reshape(NW * CPW, CHUNK)
    ue, me = _sc_gather(uid, mid, user_table, movie_table)
    return _mlp(ue, me, W1[:D], W1[D:], b1.reshape(1, 256),
                W2, b2.reshape(1, 64), W3, b3.reshape(1, 1))


# speedup vs baseline: 1.2931x; 1.2931x over previous
import functools

import jax
import jax.numpy as jnp
from jax import lax
from jax.experimental import pallas as pl
from jax.experimental.pallas import tpu as pltpu
from jax.experimental.pallas import tpu_sc as plsc

B = 16384
D = 32
NC, NS = 2, 16
NW = NC * NS
B_PER_W = B // NW
CHUNK = 128
CPW = B_PER_W // CHUNK

_sc_mesh = plsc.VectorSubcoreMesh(core_axis_name="c", subcore_axis_name="s")


@functools.partial(
    pl.kernel,
    out_type=(
        jax.ShapeDtypeStruct((B, D), jnp.float32),
        jax.ShapeDtypeStruct((B, D), jnp.float32),
    ),
    mesh=_sc_mesh,
    scratch_types=[
        pltpu.VMEM((CPW, CHUNK), jnp.int32),
        pltpu.VMEM((CPW, CHUNK), jnp.int32),
        pltpu.VMEM((B_PER_W, D), jnp.float32),
        pltpu.VMEM((B_PER_W, D), jnp.float32),
        pltpu.SemaphoreType.DMA,
    ],
    compiler_params=pltpu.CompilerParams(use_tc_tiling_on_sc=False),
)
def _sc_gather(uid_hbm, mid_hbm, ut_hbm, mt_hbm, ue_hbm, me_hbm,
               idx_u, idx_m, rows_u, rows_m, sem):
    wid = lax.axis_index("s") * NC + lax.axis_index("c")
    base = wid * CPW
    pltpu.sync_copy(uid_hbm.at[pl.ds(base, CPW)], idx_u)
    pltpu.sync_copy(mid_hbm.at[pl.ds(base, CPW)], idx_m)
    copies = []
    for j in range(CPW):
        copies.append(pltpu.async_copy(
            ut_hbm.at[idx_u.at[j]], rows_u.at[pl.ds(j * CHUNK, CHUNK)], sem))
        copies.append(pltpu.async_copy(
            mt_hbm.at[idx_m.at[j]], rows_m.at[pl.ds(j * CHUNK, CHUNK)], sem))
    for c in copies:
        c.wait()
    row0 = wid * B_PER_W
    pltpu.sync_copy(rows_u, ue_hbm.at[pl.ds(row0, B_PER_W)])
    pltpu.sync_copy(rows_m, me_hbm.at[pl.ds(row0, B_PER_W)])


BLK = 2048


def _mlp_body(ue_ref, me_ref, w1u_ref, w1m_ref, b1_ref, w2_ref, b2_ref,
              w3_ref, b3_ref, out_ref):
    x1 = jnp.dot(ue_ref[...], w1u_ref[...], preferred_element_type=jnp.float32)
    x2 = jnp.dot(me_ref[...], w1m_ref[...], preferred_element_type=jnp.float32)
    h1 = jnp.maximum(x1 + x2 + b1_ref[...], 0.0)
    h2 = jnp.maximum(
        jnp.dot(h1, w2_ref[...], preferred_element_type=jnp.float32)
        + b2_ref[...], 0.0)
    out_ref[...] = (
        jnp.dot(h2, w3_ref[...], preferred_element_type=jnp.float32)
        + b3_ref[...])


def _mlp(ue, me, w1u, w1m, b1, w2, b2, w3, b3):
    grid = (B // BLK,)
    fixed = lambda shape: pl.BlockSpec(shape, lambda i: (0, 0))
    return pl.pallas_call(
        _mlp_body,
        grid=grid,
        in_specs=[
            pl.BlockSpec((BLK, D), lambda i: (i, 0)),
            pl.BlockSpec((BLK, D), lambda i: (i, 0)),
            fixed((D, 256)),
            fixed((D, 256)),
            fixed((1, 256)),
            fixed((256, 64)),
            fixed((1, 64)),
            fixed((64, 1)),
            fixed((1, 1)),
        ],
        out_specs=pl.BlockSpec((BLK, 1), lambda i: (i, 0)),
        out_shape=jax.ShapeDtypeStruct((B, 1), jnp.float32),
    )(ue, me, w1u, w1m, b1, w2, b2, w3, b3)


def kernel(user_id, movie_title, user_table, movie_table,
           W1, b1, W2, b2, W3, b3):
    uid = user_id.astype(jnp.int32).reshape(NW * CPW, CHUNK)
    mid = movie_title.astype(jnp.int32).reshape(NW * CPW, CHUNK)
    ue, me = _sc_gather(uid, mid, user_table, movie_table)
    return _mlp(ue, me, W1[:D], W1[D:], b1.reshape(1, 256),
                W2, b2.reshape(1, 64), W3, b3.reshape(1, 1))
